# baseline (speedup 1.0000x reference)
import functools

import jax
import jax.numpy as jnp
from jax import lax
from jax.experimental import pallas as pl
from jax.experimental.pallas import tpu as pltpu
from jax.experimental.pallas import tpu_sc as plsc

_B = 16384
_D = 128
_NW = 32
_ROWS_PER_W = _B // _NW
_BLK = 128
_NBLK = _ROWS_PER_W // _BLK


def _sc_dots(u2, i2, j2, user_table, item_table):
    mesh = plsc.VectorSubcoreMesh(core_axis_name="c", subcore_axis_name="s")

    @functools.partial(
        pl.kernel,
        out_type=(
            jax.ShapeDtypeStruct((_B,), jnp.float32),
            jax.ShapeDtypeStruct((_B,), jnp.float32),
        ),
        mesh=mesh,
        scratch_types=[
            pltpu.VMEM((_NBLK, _BLK), jnp.int32),
            pltpu.VMEM((_NBLK, _BLK), jnp.int32),
            pltpu.VMEM((_NBLK, _BLK), jnp.int32),
            pltpu.VMEM((_BLK, _D), jnp.float32),
            pltpu.VMEM((_BLK, _D), jnp.float32),
            pltpu.VMEM((_BLK, _D), jnp.float32),
            pltpu.VMEM((_ROWS_PER_W,), jnp.float32),
            pltpu.VMEM((_ROWS_PER_W,), jnp.float32),
            pltpu.SemaphoreType.DMA,
        ],
        compiler_params=pltpu.CompilerParams(needs_layout_passes=False),
    )
    def k(u_hbm, i_hbm, j_hbm, ut_hbm, it_hbm, pi_hbm, pj_hbm,
          idx_u, idx_i, idx_j, ue_v, ie_v, je_v, oi_v, oj_v, sem):
        wid = lax.axis_index("s") * 2 + lax.axis_index("c")
        pltpu.sync_copy(u_hbm.at[pl.ds(wid * _NBLK, _NBLK)], idx_u)
        pltpu.sync_copy(i_hbm.at[pl.ds(wid * _NBLK, _NBLK)], idx_i)
        pltpu.sync_copy(j_hbm.at[pl.ds(wid * _NBLK, _NBLK)], idx_j)

        lanes = lax.iota(jnp.int32, 16)
        zv = jnp.zeros((16,), jnp.float32)
        for b in range(_NBLK):
            c1 = pltpu.async_copy(ut_hbm.at[idx_u.at[b]], ue_v, sem)
            c2 = pltpu.async_copy(it_hbm.at[idx_i.at[b]], ie_v, sem)
            c3 = pltpu.async_copy(it_hbm.at[idx_j.at[b]], je_v, sem)
            c1.wait()
            c2.wait()
            c3.wait()

            def grp_body(g, _, b=b):
                row_idx = g * 16 + lanes

                def d_body(d, carry):
                    ai, aj = carry
                    col = jnp.full((16,), d, jnp.int32)
                    ue = plsc.load_gather(ue_v, [row_idx, col])
                    ie = plsc.load_gather(ie_v, [row_idx, col])
                    je = plsc.load_gather(je_v, [row_idx, col])
                    return (ai + ue * ie, aj + ue * je)

                ai, aj = lax.fori_loop(0, _D, d_body, (zv, zv), unroll=8)
                oi_v[pl.ds(b * _BLK + g * 16, 16)] = ai
                oj_v[pl.ds(b * _BLK + g * 16, 16)] = aj
                return 0

            lax.fori_loop(0, _BLK // 16, grp_body, 0)

        pltpu.sync_copy(oi_v, pi_hbm.at[pl.ds(wid * _ROWS_PER_W, _ROWS_PER_W)])
        pltpu.sync_copy(oj_v, pj_hbm.at[pl.ds(wid * _ROWS_PER_W, _ROWS_PER_W)])

    return k(u2, i2, j2, user_table, item_table)


def _tc_loss_body(pi_ref, pj_ref, out_ref):
    x = pi_ref[...]
    y = pj_ref[...]

    def softplus(t):
        return jnp.maximum(t, 0.0) + jnp.log1p(jnp.exp(-jnp.abs(t)))

    out_ref[0, 0] = jnp.sum(softplus(-x)) + jnp.sum(softplus(y))


def kernel(u, i, j, user_table, item_table):
    u2 = u.reshape(_NW * _NBLK, _BLK).astype(jnp.int32)
    i2 = i.reshape(_NW * _NBLK, _BLK).astype(jnp.int32)
    j2 = j.reshape(_NW * _NBLK, _BLK).astype(jnp.int32)
    pred_i, pred_j = _sc_dots(u2, i2, j2, user_table, item_table)

    loss = pl.pallas_call(
        _tc_loss_body,
        out_shape=jax.ShapeDtypeStruct((1, 1), jnp.float32),
        out_specs=pl.BlockSpec(memory_space=pltpu.SMEM),
    )(pred_i.reshape(_BLK, _BLK), pred_j.reshape(_BLK, _BLK))
    return loss[0, 0]


# speedup vs baseline: 3.0065x; 3.0065x over previous
import functools

import jax
import jax.numpy as jnp
from jax import lax
from jax.experimental import pallas as pl
from jax.experimental.pallas import tpu as pltpu
from jax.experimental.pallas import tpu_sc as plsc

_B = 16384
_D = 128
_NW = 32
_ROWS_PER_W = _B // _NW
_BLK = 128
_NBLK = _ROWS_PER_W // _BLK
_PAD = 17


def _sc_dots(u2, i2, j2, user_table, item_table):
    mesh = plsc.VectorSubcoreMesh(core_axis_name="c", subcore_axis_name="s")

    @functools.partial(
        pl.kernel,
        out_type=(
            jax.ShapeDtypeStruct((_B,), jnp.float32),
            jax.ShapeDtypeStruct((_B,), jnp.float32),
        ),
        mesh=mesh,
        scratch_types=[
            pltpu.VMEM((_NBLK, _BLK), jnp.int32),
            pltpu.VMEM((_NBLK, _BLK), jnp.int32),
            pltpu.VMEM((_NBLK, _BLK), jnp.int32),
            pltpu.VMEM((_BLK, _D), jnp.float32),
            pltpu.VMEM((_BLK, _D), jnp.float32),
            pltpu.VMEM((_BLK, _D), jnp.float32),
            pltpu.VMEM((_BLK, _D), jnp.float32),
            pltpu.VMEM((_BLK, _D), jnp.float32),
            pltpu.VMEM((_BLK, _D), jnp.float32),
            pltpu.VMEM((16 * _PAD,), jnp.float32),
            pltpu.VMEM((16 * _PAD,), jnp.float32),
            pltpu.VMEM((_ROWS_PER_W,), jnp.float32),
            pltpu.VMEM((_ROWS_PER_W,), jnp.float32),
            pltpu.SemaphoreType.DMA,
            pltpu.SemaphoreType.DMA,
        ],
        compiler_params=pltpu.CompilerParams(needs_layout_passes=False),
    )
    def k(u_hbm, i_hbm, j_hbm, ut_hbm, it_hbm, pi_hbm, pj_hbm,
          idx_u, idx_i, idx_j, ue0, ie0, je0, ue1, ie1, je1,
          tb_i, tb_j, oi_v, oj_v, sem0, sem1):
        wid = lax.axis_index("s") * 2 + lax.axis_index("c")
        pltpu.sync_copy(u_hbm.at[pl.ds(wid * _NBLK, _NBLK)], idx_u)
        pltpu.sync_copy(i_hbm.at[pl.ds(wid * _NBLK, _NBLK)], idx_i)
        pltpu.sync_copy(j_hbm.at[pl.ds(wid * _NBLK, _NBLK)], idx_j)

        bufs = ((ue0, ie0, je0, sem0), (ue1, ie1, je1, sem1))

        def issue(b):
            ue_v, ie_v, je_v, sem = bufs[b % 2]
            return (
                pltpu.async_copy(ut_hbm.at[idx_u.at[b]], ue_v, sem),
                pltpu.async_copy(it_hbm.at[idx_i.at[b]], ie_v, sem),
                pltpu.async_copy(it_hbm.at[idx_j.at[b]], je_v, sem),
            )

        lanes = lax.iota(jnp.int32, 16)
        lanes17 = lanes * _PAD
        zv = jnp.zeros((16,), jnp.float32)

        pending = issue(0)
        for b in range(_NBLK):
            for c in pending:
                c.wait()
            if b + 1 < _NBLK:
                pending = issue(b + 1)
            ue_v, ie_v, je_v, _ = bufs[b % 2]

            def grp_body(g, _, b=b):
                r0 = g * 16
                for r in range(16):
                    acc_i = zv
                    acc_j = zv
                    for c in range(_D // 16):
                        ue = ue_v[r0 + r, pl.ds(c * 16, 16)]
                        ie = ie_v[r0 + r, pl.ds(c * 16, 16)]
                        je = je_v[r0 + r, pl.ds(c * 16, 16)]
                        acc_i = acc_i + ue * ie
                        acc_j = acc_j + ue * je
                    row_idx = lanes + (r * _PAD)
                    plsc.store_scatter(tb_i, [row_idx], acc_i)
                    plsc.store_scatter(tb_j, [row_idx], acc_j)
                s_i = zv
                s_j = zv
                for c in range(16):
                    s_i = s_i + plsc.load_gather(tb_i, [lanes17 + c])
                    s_j = s_j + plsc.load_gather(tb_j, [lanes17 + c])
                oi_v[pl.ds(b * _BLK + r0, 16)] = s_i
                oj_v[pl.ds(b * _BLK + r0, 16)] = s_j
                return 0

            lax.fori_loop(0, _BLK // 16, grp_body, 0)

        pltpu.sync_copy(oi_v, pi_hbm.at[pl.ds(wid * _ROWS_PER_W, _ROWS_PER_W)])
        pltpu.sync_copy(oj_v, pj_hbm.at[pl.ds(wid * _ROWS_PER_W, _ROWS_PER_W)])

    return k(u2, i2, j2, user_table, item_table)


def _tc_loss_body(pi_ref, pj_ref, out_ref):
    x = pi_ref[...]
    y = pj_ref[...]

    def softplus(t):
        return jnp.maximum(t, 0.0) + jnp.log1p(jnp.exp(-jnp.abs(t)))

    out_ref[0, 0] = jnp.sum(softplus(-x)) + jnp.sum(softplus(y))


def kernel(u, i, j, user_table, item_table):
    u2 = u.reshape(_NW * _NBLK, _BLK).astype(jnp.int32)
    i2 = i.reshape(_NW * _NBLK, _BLK).astype(jnp.int32)
    j2 = j.reshape(_NW * _NBLK, _BLK).astype(jnp.int32)
    pred_i, pred_j = _sc_dots(u2, i2, j2, user_table, item_table)

    loss = pl.pallas_call(
        _tc_loss_body,
        out_shape=jax.ShapeDtypeStruct((1, 1), jnp.float32),
        out_specs=pl.BlockSpec(memory_space=pltpu.SMEM),
    )(pred_i.reshape(_BLK, _BLK), pred_j.reshape(_BLK, _BLK))
    return loss[0, 0]


# speedup vs baseline: 3.2493x; 1.0808x over previous
import functools

import jax
import jax.numpy as jnp
from jax import lax
from jax.experimental import pallas as pl
from jax.experimental.pallas import tpu as pltpu
from jax.experimental.pallas import tpu_sc as plsc

_B = 16384
_D = 128
_NW = 32
_ROWS_PER_W = _B // _NW
_BLK = 128
_NBLK = _ROWS_PER_W // _BLK
_PAD = 17


def _sc_dots(idx, user_table, item_table):
    mesh = plsc.VectorSubcoreMesh(core_axis_name="c", subcore_axis_name="s")

    @functools.partial(
        pl.kernel,
        out_type=(
            jax.ShapeDtypeStruct((_B,), jnp.float32),
            jax.ShapeDtypeStruct((_B,), jnp.float32),
        ),
        mesh=mesh,
        scratch_types=[
            pltpu.VMEM((_NBLK, 3, _BLK), jnp.int32),
            pltpu.VMEM((_BLK, _D), jnp.float32),
            pltpu.VMEM((_BLK, _D), jnp.float32),
            pltpu.VMEM((_BLK, _D), jnp.float32),
            pltpu.VMEM((_BLK, _D), jnp.float32),
            pltpu.VMEM((_BLK, _D), jnp.float32),
            pltpu.VMEM((_BLK, _D), jnp.float32),
            pltpu.VMEM((16 * _PAD,), jnp.float32),
            pltpu.VMEM((16 * _PAD,), jnp.float32),
            pltpu.VMEM((_ROWS_PER_W,), jnp.float32),
            pltpu.VMEM((_ROWS_PER_W,), jnp.float32),
            pltpu.SemaphoreType.DMA,
            pltpu.SemaphoreType.DMA,
            pltpu.SemaphoreType.DMA,
        ],
        compiler_params=pltpu.CompilerParams(needs_layout_passes=False),
    )
    def k(idx_hbm, ut_hbm, it_hbm, pi_hbm, pj_hbm,
          idx_v, ue0, ie0, je0, ue1, ie1, je1,
          tb_i, tb_j, oi_v, oj_v, sem0, sem1, osem):
        wid = lax.axis_index("s") * 2 + lax.axis_index("c")
        pltpu.sync_copy(idx_hbm.at[pl.ds(wid * _NBLK, _NBLK)], idx_v)

        sets = ((ue0, ie0, je0, sem0), (ue1, ie1, je1, sem1))

        def copies(b, sub):
            ue_v, ie_v, je_v, sem = sets[sub]
            return (
                pltpu.make_async_copy(ut_hbm.at[idx_v.at[b, 0]], ue_v, sem),
                pltpu.make_async_copy(it_hbm.at[idx_v.at[b, 1]], ie_v, sem),
                pltpu.make_async_copy(it_hbm.at[idx_v.at[b, 2]], je_v, sem),
            )

        for c in copies(0, 0):
            c.start()
        for c in copies(1, 1):
            c.start()

        lanes = lax.iota(jnp.int32, 16)
        lanes17 = lanes * _PAD
        zv = jnp.zeros((16,), jnp.float32)

        def pair_body(p, _):
            for sub in range(2):
                b = 2 * p + sub
                ue_v, ie_v, je_v, _sem = sets[sub]
                for c in copies(b, sub):
                    c.wait()

                def grp_body(g, _, ue_v=ue_v, ie_v=ie_v, je_v=je_v, b=b):
                    r0 = g * 16

                    def row_body(r, _):
                        acc_i = zv
                        acc_j = zv
                        for c in range(_D // 16):
                            ue = ue_v[r0 + r, pl.ds(c * 16, 16)]
                            ie = ie_v[r0 + r, pl.ds(c * 16, 16)]
                            je = je_v[r0 + r, pl.ds(c * 16, 16)]
                            acc_i = acc_i + ue * ie
                            acc_j = acc_j + ue * je
                        row_idx = lanes + r * _PAD
                        plsc.store_scatter(tb_i, [row_idx], acc_i)
                        plsc.store_scatter(tb_j, [row_idx], acc_j)
                        return 0

                    lax.fori_loop(0, 16, row_body, 0, unroll=8)
                    s_i = zv
                    s_j = zv
                    for c in range(16):
                        s_i = s_i + plsc.load_gather(tb_i, [lanes17 + c])
                        s_j = s_j + plsc.load_gather(tb_j, [lanes17 + c])
                    oi_v[pl.ds(b * _BLK + r0, 16)] = s_i
                    oj_v[pl.ds(b * _BLK + r0, 16)] = s_j
                    return 0

                lax.fori_loop(0, _BLK // 16, grp_body, 0)

                @pl.when(b + 2 < _NBLK)
                def _():
                    for c in copies(b + 2, sub):
                        c.start()
            return 0

        lax.fori_loop(0, _NBLK // 2, pair_body, 0)

        o1 = pltpu.make_async_copy(
            oi_v, pi_hbm.at[pl.ds(wid * _ROWS_PER_W, _ROWS_PER_W)], osem)
        o2 = pltpu.make_async_copy(
            oj_v, pj_hbm.at[pl.ds(wid * _ROWS_PER_W, _ROWS_PER_W)], osem)
        o1.start()
        o2.start()
        o1.wait()
        o2.wait()

    return k(idx, user_table, item_table)


def _tc_loss_body(pi_ref, pj_ref, out_ref):
    x = pi_ref[...]
    y = pj_ref[...]

    def softplus(t):
        return jnp.maximum(t, 0.0) + jnp.log1p(jnp.exp(-jnp.abs(t)))

    out_ref[0, 0] = jnp.sum(softplus(-x)) + jnp.sum(softplus(y))


def kernel(u, i, j, user_table, item_table):
    idx = jnp.stack(
        [x.reshape(_NW * _NBLK, _BLK).astype(jnp.int32) for x in (u, i, j)],
        axis=1,
    )
    pred_i, pred_j = _sc_dots(idx, user_table, item_table)

    loss = pl.pallas_call(
        _tc_loss_body,
        out_shape=jax.ShapeDtypeStruct((1, 1), jnp.float32),
        out_specs=pl.BlockSpec(memory_space=pltpu.SMEM),
    )(pred_i.reshape(_BLK, _BLK), pred_j.reshape(_BLK, _BLK))
    return loss[0, 0]
